# baseline (speedup 1.0000x reference)
import functools

import jax
import jax.numpy as jnp
from jax import lax
from jax.experimental import pallas as pl
from jax.experimental.pallas import tpu as pltpu
from jax.experimental.pallas import tpu_sc as plsc

VOCAB = 1000000
EMB = 16
BATCH = 16384

_NC = 2
_NS = 16
_NW = _NC * _NS
_B_PER_W = BATCH // _NW
_NJ = 7813
_GROWS = _NJ * 16 + 16
_GWORDS = _GROWS * 128

_mesh = plsc.VectorSubcoreMesh(core_axis_name="c", subcore_axis_name="s")


@functools.partial(
    pl.kernel,
    mesh=_mesh,
    out_type=jax.ShapeDtypeStruct((_GWORDS,), jnp.float32),
    scratch_types=[
        pltpu.VMEM((EMB, 128), jnp.float32),
        pltpu.VMEM((EMB, 128), jnp.float32),
        pltpu.VMEM((2048,), jnp.float32),
        pltpu.VMEM((2048,), jnp.float32),
        pltpu.SemaphoreType.DMA,
        pltpu.SemaphoreType.DMA,
        pltpu.SemaphoreType.DMA,
        pltpu.SemaphoreType.DMA,
    ],
    compiler_params=pltpu.CompilerParams(needs_layout_passes=False),
)
def _repack(tT_hbm, g_hbm, in0, in1, out0, out1, semA, semB, semW0, semW1):
    wid = lax.axis_index("s") * _NC + lax.axis_index("c")
    iota16 = lax.iota(jnp.int32, 16)

    addrbase = [(c0 + iota16) * 16 for c0 in range(0, 128, 16)]

    def jof(t):
        return wid + _NW * t

    def fetch(t, blk, sem):
        pltpu.async_copy(tT_hbm.at[:, pl.ds(jof(t) * 128, 128)], blk, sem)

    def wait_in(blk, sem):
        pltpu.make_async_copy(tT_hbm.at[:, pl.ds(0, 128)], blk, sem).wait()

    def wait_out(blk, sem):
        pltpu.make_async_copy(blk, g_hbm.at[pl.ds(0, 2048)], sem).wait()

    def shuffle(src, dst):
        @plsc.parallel_loop(0, 8, 1, unroll=4)
        def _(b):
            ab = (b * 16 + iota16) * 16
            for d in range(16):
                vals = src[d, pl.ds(b * 16, 16)]
                plsc.store_scatter(dst, [ab + d], vals)

    def store(t, blk, sem):
        pltpu.async_copy(blk, g_hbm.at[pl.ds(jof(t) * 2048, 2048)], sem)

    pltpu.async_copy(out0, g_hbm.at[pl.ds(_NJ * 2048, 2048)], semW0)
    pltpu.async_copy(out1, g_hbm.at[pl.ds(_NJ * 2048, 2048)], semW1)
    fetch(0, in0, semA)

    def body(i, carry):
        t0 = 2 * i
        fetch(t0 + 1, in1, semB)
        wait_in(in0, semA)
        wait_out(out0, semW0)
        shuffle(in0, out0)
        store(t0, out0, semW0)

        @pl.when(jof(t0 + 2) < _NJ)
        def _():
            fetch(t0 + 2, in0, semA)

        wait_in(in1, semB)
        wait_out(out1, semW1)
        shuffle(in1, out1)
        store(t0 + 1, out1, semW1)
        return carry

    lax.fori_loop(0, 122, body, 0)

    @pl.when(jof(244) < _NJ)
    def _():
        wait_in(in0, semA)
        wait_out(out0, semW0)
        shuffle(in0, out0)
        store(244, out0, semW0)

    wait_out(out0, semW0)
    wait_out(out1, semW1)


@functools.partial(
    pl.kernel,
    mesh=_mesh,
    out_type=jax.ShapeDtypeStruct((EMB, BATCH), jnp.float32),
    scratch_types=[
        pltpu.VMEM((_B_PER_W,), jnp.int32),
        pltpu.VMEM((_B_PER_W,), jnp.int32),
        pltpu.VMEM((_B_PER_W, 128), jnp.float32),
        pltpu.VMEM((EMB, _B_PER_W), jnp.float32),
        pltpu.SemaphoreType.DMA,
    ],
    compiler_params=pltpu.CompilerParams(needs_layout_passes=False),
)
def _gather(idx_hbm, g_hbm, outT_hbm, idx_v, pv, rows_v, out_v, sem):
    wid = lax.axis_index("s") * _NC + lax.axis_index("c")
    base = wid * _B_PER_W
    iota16 = lax.iota(jnp.int32, 16)

    pltpu.sync_copy(idx_hbm.at[pl.ds(base, _B_PER_W)], idx_v)

    def prep(g, carry):
        rv = idx_v[pl.ds(g * 16, 16)]
        pv[pl.ds(g * 16, 16)] = lax.shift_right_logical(rv, 3)
        return carry

    lax.fori_loop(0, _B_PER_W // 16, prep, 0)

    pltpu.async_copy(g_hbm.at[pv], rows_v, sem).wait()

    @plsc.parallel_loop(0, _B_PER_W // 16, 1, unroll=2)
    def _(g):
        rv = idx_v[pl.ds(g * 16, 16)]
        lanev = (rv & 7) * 16
        for j in range(16):
            k = g * 16 + j
            kv = jnp.full((16,), k, jnp.int32)
            vals = plsc.load_gather(rows_v, [kv, lanev[j] + iota16])
            plsc.store_scatter(out_v, [iota16, kv], vals)

    pltpu.sync_copy(out_v, outT_hbm.at[:, pl.ds(base, _B_PER_W)])


def kernel(indices, table):
    g = _repack(table.T)
    outT = _gather(indices.astype(jnp.int32), g.reshape(_GROWS, 128))
    return outT.T


# speedup vs baseline: 1.1421x; 1.1421x over previous
import functools

import jax
import jax.numpy as jnp
from jax import lax
from jax.experimental import pallas as pl
from jax.experimental.pallas import tpu as pltpu
from jax.experimental.pallas import tpu_sc as plsc

VOCAB = 1000000
EMB = 16
BATCH = 16384

_NC = 2
_NS = 16
_NW = _NC * _NS
_B_PER_W = BATCH // _NW
_NJ = 7813
_GROWS = _NJ * 16 + 16
_GWORDS = _GROWS * 128

_mesh = plsc.VectorSubcoreMesh(core_axis_name="c", subcore_axis_name="s")


@functools.partial(
    pl.kernel,
    mesh=_mesh,
    out_type=jax.ShapeDtypeStruct((_GWORDS,), jnp.float32),
    scratch_types=[
        pltpu.VMEM((EMB, 128), jnp.float32),
        pltpu.VMEM((EMB, 128), jnp.float32),
        pltpu.VMEM((EMB, 128), jnp.float32),
        pltpu.VMEM((EMB, 128), jnp.float32),
        pltpu.VMEM((2048,), jnp.float32),
        pltpu.VMEM((2048,), jnp.float32),
        pltpu.VMEM((2048,), jnp.float32),
        pltpu.VMEM((2048,), jnp.float32),
        pltpu.SemaphoreType.DMA,
        pltpu.SemaphoreType.DMA,
        pltpu.SemaphoreType.DMA,
        pltpu.SemaphoreType.DMA,
        pltpu.SemaphoreType.DMA,
        pltpu.SemaphoreType.DMA,
        pltpu.SemaphoreType.DMA,
        pltpu.SemaphoreType.DMA,
    ],
    compiler_params=pltpu.CompilerParams(needs_layout_passes=False),
)
def _repack(tT_hbm, g_hbm, in0, in1, in2, in3, out0, out1, out2, out3,
            sI0, sI1, sI2, sI3, sW0, sW1, sW2, sW3):
    wid = lax.axis_index("s") * _NC + lax.axis_index("c")
    iota16 = lax.iota(jnp.int32, 16)
    ins = [in0, in1, in2, in3]
    outs = [out0, out1, out2, out3]
    sIs = [sI0, sI1, sI2, sI3]
    sWs = [sW0, sW1, sW2, sW3]

    def jof(t):
        return wid + _NW * t

    def fetch(t, blk, sem):
        pltpu.async_copy(tT_hbm.at[:, pl.ds(jof(t) * 128, 128)], blk, sem)

    def wait_in(blk, sem):
        pltpu.make_async_copy(tT_hbm.at[:, pl.ds(0, 128)], blk, sem).wait()

    def wait_out(blk, sem):
        pltpu.make_async_copy(blk, g_hbm.at[pl.ds(0, 2048)], sem).wait()

    def shuffle(src, dst):
        @plsc.parallel_loop(0, 8, 1, unroll=4)
        def _(b):
            ab = (b * 16 + iota16) * 16
            for d in range(16):
                vals = src[d, pl.ds(b * 16, 16)]
                plsc.store_scatter(dst, [ab + d], vals)

    def store(t, blk, sem):
        pltpu.async_copy(blk, g_hbm.at[pl.ds(jof(t) * 2048, 2048)], sem)

    for s in range(4):
        pltpu.async_copy(outs[s], g_hbm.at[pl.ds(_NJ * 2048, 2048)], sWs[s])
    for t in range(3):
        fetch(t, ins[t], sIs[t])

    def step(t, s):
        @pl.when(jof(t + 3) < _NJ)
        def _():
            fetch(t + 3, ins[(s + 3) % 4], sIs[(s + 3) % 4])

        wait_in(ins[s], sIs[s])
        wait_out(outs[s], sWs[s])
        shuffle(ins[s], outs[s])
        store(t, outs[s], sWs[s])

    def body(i, carry):
        for s in range(4):
            step(4 * i + s, s)
        return carry

    lax.fori_loop(0, 61, body, 0)

    @pl.when(jof(244) < _NJ)
    def _():
        wait_in(in0, sI0)
        wait_out(out0, sW0)
        shuffle(in0, out0)
        store(244, out0, sW0)

    for s in range(4):
        wait_out(outs[s], sWs[s])


@functools.partial(
    pl.kernel,
    mesh=_mesh,
    out_type=jax.ShapeDtypeStruct((EMB, BATCH), jnp.float32),
    scratch_types=[
        pltpu.VMEM((_B_PER_W,), jnp.int32),
        pltpu.VMEM((_B_PER_W,), jnp.int32),
        pltpu.VMEM((_B_PER_W, 128), jnp.float32),
        pltpu.VMEM((EMB, _B_PER_W), jnp.float32),
        pltpu.SemaphoreType.DMA,
    ],
    compiler_params=pltpu.CompilerParams(needs_layout_passes=False),
)
def _gather(idx_hbm, g_hbm, outT_hbm, idx_v, pv, rows_v, out_v, sem):
    wid = lax.axis_index("s") * _NC + lax.axis_index("c")
    base = wid * _B_PER_W
    iota16 = lax.iota(jnp.int32, 16)

    pltpu.sync_copy(idx_hbm.at[pl.ds(base, _B_PER_W)], idx_v)

    def prep(g, carry):
        rv = idx_v[pl.ds(g * 16, 16)]
        pv[pl.ds(g * 16, 16)] = lax.shift_right_logical(rv, 3)
        return carry

    lax.fori_loop(0, _B_PER_W // 16, prep, 0)

    pltpu.async_copy(g_hbm.at[pv], rows_v, sem).wait()

    @plsc.parallel_loop(0, _B_PER_W // 16, 1, unroll=2)
    def _(g):
        rv = idx_v[pl.ds(g * 16, 16)]
        lanev = (rv & 7) * 16
        for j in range(16):
            k = g * 16 + j
            kv = jnp.full((16,), k, jnp.int32)
            vals = plsc.load_gather(rows_v, [kv, lanev[j] + iota16])
            plsc.store_scatter(out_v, [iota16, kv], vals)

    pltpu.sync_copy(out_v, outT_hbm.at[:, pl.ds(base, _B_PER_W)])


def kernel(indices, table):
    g = _repack(table.T)
    outT = _gather(indices.astype(jnp.int32), g.reshape(_GROWS, 128))
    return outT.T
